# baseline (speedup 1.0000x reference)
import jax
import jax.numpy as jnp
from jax import lax
from jax.experimental import pallas as pl
from jax.experimental.pallas import tpu as pltpu
from jax.experimental.pallas import tpu_sc as plsc

DIM = 256
HALF = DIM // 2
PAD = 144
BATCH = 4096
NPAIRS = 64
NW = 32
L = 16
NG = HALF // L
UNROLL = 2

TC_ROWS = 2048
TC_TILE = 64
SC_ROWS = BATCH - TC_ROWS
ROWS_PER_W = SC_ROWS // NW


def _trig_body(te_ref, top_ref, ce_ref, se_ref, co_ref, so_ref):
    ce_ref[...] = jnp.cos(2.0 * te_ref[...])
    se_ref[...] = jnp.sin(2.0 * te_ref[...])
    co_ref[...] = jnp.cos(2.0 * top_ref[...])
    so_ref[...] = jnp.sin(2.0 * top_ref[...])


_trig = pl.pallas_call(
    _trig_body,
    out_shape=[jax.ShapeDtypeStruct((NPAIRS, HALF), jnp.float32)] * 4,
)


def _take(v, idx):
    return lax.gather(
        v, idx[:, None],
        lax.GatherDimensionNumbers(offset_dims=(), collapsed_slice_dims=(0,),
                                   start_index_map=(0,)),
        slice_sizes=(1,),
        mode=lax.GatherScatterMode.PROMISE_IN_BOUNDS)


def _clements_body(x_hbm, ce_hbm, se_hbm, co_hbm, so_hbm, out_hbm,
                   X, A, B, CE, SE, CO, SO):
    wid = lax.axis_index("s") * 2 + lax.axis_index("c")
    base = wid * ROWS_PER_W
    pltpu.sync_copy(x_hbm.at[pl.ds(base, ROWS_PER_W)], X)
    pltpu.sync_copy(ce_hbm, CE)
    pltpu.sync_copy(se_hbm, SE)
    pltpu.sync_copy(co_hbm, CO)
    pltpu.sync_copy(so_hbm, SO)

    iota = lax.iota(jnp.int32, L)
    idx_a = (2 * iota) & 15
    idx_b = (2 * iota + 1) & 15
    lo_half = iota < 8
    zeros = jnp.zeros((L,), jnp.float32)

    @plsc.parallel_loop(0, ROWS_PER_W, unroll=UNROLL)
    def _deinterleave(r):
        for j in range(NG):
            v0 = X[r, pl.ds(32 * j, L)]
            v1 = X[r, pl.ds(32 * j + 16, L)]
            A[r, pl.ds(L * j, L)] = jnp.where(
                lo_half, _take(v0, idx_a), _take(v1, idx_a))
            B[r, pl.ds(L * j, L)] = jnp.where(
                lo_half, _take(v0, idx_b), _take(v1, idx_b))
        A[r, pl.ds(HALF, L)] = zeros

    def layer_pair(l, carry):
        ces = [CE[l, pl.ds(L * g, L)] for g in range(NG)]
        ses = [SE[l, pl.ds(L * g, L)] for g in range(NG)]

        @plsc.parallel_loop(0, ROWS_PER_W, unroll=UNROLL)
        def _even(r):
            for g in range(NG):
                sl = pl.ds(L * g, L)
                a = A[r, sl]
                b = B[r, sl]
                A[r, sl] = a * ces[g] + b * ses[g]
                B[r, sl] = a * ses[g] - b * ces[g]

        cos_ = [CO[l, pl.ds(L * g, L)] for g in range(NG)]
        sos = [SO[l, pl.ds(L * g, L)] for g in range(NG)]

        @plsc.parallel_loop(0, ROWS_PER_W, unroll=UNROLL)
        def _odd(r):
            for g in range(NG):
                sl = pl.ds(L * g, L)
                sl1 = pl.ds(L * g + 1, L)
                b = B[r, sl]
                a1 = A[r, sl1]
                B[r, sl] = b * cos_[g] + a1 * sos[g]
                A[r, sl1] = b * sos[g] - a1 * cos_[g]

        return carry

    lax.fori_loop(0, NPAIRS, layer_pair, 0)

    idx_h = iota >> 1
    even_lane = (iota & 1) == 0

    @plsc.parallel_loop(0, ROWS_PER_W, unroll=UNROLL)
    def _interleave(r):
        for j in range(NG):
            a = A[r, pl.ds(L * j, L)]
            b = B[r, pl.ds(L * j, L)]
            X[r, pl.ds(32 * j, L)] = jnp.where(
                even_lane, _take(a, idx_h), _take(b, idx_h))
            X[r, pl.ds(32 * j + 16, L)] = jnp.where(
                even_lane, _take(a, 8 + idx_h), _take(b, 8 + idx_h))

    pltpu.sync_copy(X, out_hbm.at[pl.ds(base, ROWS_PER_W)])


_clements_sc = pl.kernel(
    _clements_body,
    out_type=jax.ShapeDtypeStruct((SC_ROWS, DIM), jnp.float32),
    mesh=plsc.VectorSubcoreMesh(core_axis_name="c", subcore_axis_name="s",
                                num_cores=2, num_subcores=16),
    compiler_params=pltpu.CompilerParams(use_tc_tiling_on_sc=False),
    scratch_types=[
        pltpu.VMEM((ROWS_PER_W, DIM), jnp.float32),
        pltpu.VMEM((ROWS_PER_W, PAD), jnp.float32),
        pltpu.VMEM((ROWS_PER_W, HALF), jnp.float32),
        pltpu.VMEM((NPAIRS, HALF), jnp.float32),
        pltpu.VMEM((NPAIRS, HALF), jnp.float32),
        pltpu.VMEM((NPAIRS, HALF), jnp.float32),
        pltpu.VMEM((NPAIRS, HALF), jnp.float32),
    ],
)


def _tc_body(ce_ref, se_ref, co_ref, so_ref, a_ref, b_ref, ao_ref, bo_ref):
    a0 = a_ref[...]
    b0 = b_ref[...]
    col = lax.broadcasted_iota(jnp.int32, (TC_TILE, HALF), 1)

    def pair(l, ab):
        a, b = ab
        cev = ce_ref[l][None, :]
        sev = se_ref[l][None, :]
        na = a * cev + b * sev
        nb = a * sev - b * cev
        cov = co_ref[l][None, :]
        sov = so_ref[l][None, :]
        a1 = jnp.roll(na, -1, axis=1)
        nb2 = nb * cov + a1 * sov
        t = nb * sov - a1 * cov
        na2 = jnp.where(col == 0, na, jnp.roll(t, 1, axis=1))
        return (na2, nb2)

    a, b = lax.fori_loop(0, NPAIRS, pair, (a0, b0))
    ao_ref[...] = a
    bo_ref[...] = b


_clements_tc = pl.pallas_call(
    _tc_body,
    grid=(TC_ROWS // TC_TILE,),
    in_specs=[
        pl.BlockSpec((NPAIRS, HALF), lambda i: (0, 0)),
        pl.BlockSpec((NPAIRS, HALF), lambda i: (0, 0)),
        pl.BlockSpec((NPAIRS, HALF), lambda i: (0, 0)),
        pl.BlockSpec((NPAIRS, HALF), lambda i: (0, 0)),
        pl.BlockSpec((TC_TILE, HALF), lambda i: (i, 0)),
        pl.BlockSpec((TC_TILE, HALF), lambda i: (i, 0)),
    ],
    out_specs=[
        pl.BlockSpec((TC_TILE, HALF), lambda i: (i, 0)),
        pl.BlockSpec((TC_TILE, HALF), lambda i: (i, 0)),
    ],
    out_shape=[jax.ShapeDtypeStruct((TC_ROWS, HALF), jnp.float32)] * 2,
)


def kernel(x, thetas_even, thetas_odd):
    to_p = jnp.pad(thetas_odd, ((0, 0), (0, 1)))
    ce, se, co, so = _trig(thetas_even, to_p)
    out_sc = _clements_sc(x[TC_ROWS:], ce, se, co, so)
    xt = x[:TC_ROWS].reshape(TC_ROWS, HALF, 2)
    ta, tb = _clements_tc(ce, se, co, so, xt[:, :, 0], xt[:, :, 1])
    out_tc = jnp.stack([ta, tb], axis=-1).reshape(TC_ROWS, DIM)
    return jnp.concatenate([out_tc, out_sc], axis=0)


# speedup vs baseline: 6.1432x; 6.1432x over previous
import numpy as np

import jax
import jax.numpy as jnp
from jax import lax
from jax.experimental import pallas as pl
from jax.experimental.pallas import tpu as pltpu
from jax.experimental.pallas import tpu_sc as plsc

DIM = 256
HALF = DIM // 2
PAD = 144
BATCH = 4096
NPAIRS = 64
NW = 32
L = 16
NG = HALF // L
UNROLL = 2

SC_ROWS = DIM
ROWS_PER_W = SC_ROWS // NW
MM_TILE = 512


def _trig_body(te_ref, top_ref, ce_ref, se_ref, co_ref, so_ref):
    ce_ref[...] = jnp.cos(2.0 * te_ref[...])
    se_ref[...] = jnp.sin(2.0 * te_ref[...])
    co_ref[...] = jnp.cos(2.0 * top_ref[...])
    so_ref[...] = jnp.sin(2.0 * top_ref[...])


_trig = pl.pallas_call(
    _trig_body,
    out_shape=[jax.ShapeDtypeStruct((NPAIRS, HALF), jnp.float32)] * 4,
)


def _take(v, idx):
    return lax.gather(
        v, idx[:, None],
        lax.GatherDimensionNumbers(offset_dims=(), collapsed_slice_dims=(0,),
                                   start_index_map=(0,)),
        slice_sizes=(1,),
        mode=lax.GatherScatterMode.PROMISE_IN_BOUNDS)


def _clements_body(x_hbm, ce_hbm, se_hbm, co_hbm, so_hbm, out_hbm,
                   X, A, B, CE, SE, CO, SO):
    wid = lax.axis_index("s") * 2 + lax.axis_index("c")
    base = wid * ROWS_PER_W
    pltpu.sync_copy(x_hbm.at[pl.ds(base, ROWS_PER_W)], X)
    pltpu.sync_copy(ce_hbm, CE)
    pltpu.sync_copy(se_hbm, SE)
    pltpu.sync_copy(co_hbm, CO)
    pltpu.sync_copy(so_hbm, SO)

    iota = lax.iota(jnp.int32, L)
    idx_a = (2 * iota) & 15
    idx_b = (2 * iota + 1) & 15
    lo_half = iota < 8
    zeros = jnp.zeros((L,), jnp.float32)

    @plsc.parallel_loop(0, ROWS_PER_W, unroll=UNROLL)
    def _deinterleave(r):
        for j in range(NG):
            v0 = X[r, pl.ds(32 * j, L)]
            v1 = X[r, pl.ds(32 * j + 16, L)]
            A[r, pl.ds(L * j, L)] = jnp.where(
                lo_half, _take(v0, idx_a), _take(v1, idx_a))
            B[r, pl.ds(L * j, L)] = jnp.where(
                lo_half, _take(v0, idx_b), _take(v1, idx_b))
        A[r, pl.ds(HALF, L)] = zeros

    def layer_pair(l, carry):
        ces = [CE[l, pl.ds(L * g, L)] for g in range(NG)]
        ses = [SE[l, pl.ds(L * g, L)] for g in range(NG)]

        @plsc.parallel_loop(0, ROWS_PER_W, unroll=UNROLL)
        def _even(r):
            for g in range(NG):
                sl = pl.ds(L * g, L)
                a = A[r, sl]
                b = B[r, sl]
                A[r, sl] = a * ces[g] + b * ses[g]
                B[r, sl] = a * ses[g] - b * ces[g]

        cos_ = [CO[l, pl.ds(L * g, L)] for g in range(NG)]
        sos = [SO[l, pl.ds(L * g, L)] for g in range(NG)]

        @plsc.parallel_loop(0, ROWS_PER_W, unroll=UNROLL)
        def _odd(r):
            for g in range(NG):
                sl = pl.ds(L * g, L)
                sl1 = pl.ds(L * g + 1, L)
                b = B[r, sl]
                a1 = A[r, sl1]
                B[r, sl] = b * cos_[g] + a1 * sos[g]
                A[r, sl1] = b * sos[g] - a1 * cos_[g]

        return carry

    lax.fori_loop(0, NPAIRS, layer_pair, 0)

    idx_h = iota >> 1
    even_lane = (iota & 1) == 0

    @plsc.parallel_loop(0, ROWS_PER_W, unroll=UNROLL)
    def _interleave(r):
        for j in range(NG):
            a = A[r, pl.ds(L * j, L)]
            b = B[r, pl.ds(L * j, L)]
            X[r, pl.ds(32 * j, L)] = jnp.where(
                even_lane, _take(a, idx_h), _take(b, idx_h))
            X[r, pl.ds(32 * j + 16, L)] = jnp.where(
                even_lane, _take(a, 8 + idx_h), _take(b, 8 + idx_h))

    pltpu.sync_copy(X, out_hbm.at[pl.ds(base, ROWS_PER_W)])


_clements_sc = pl.kernel(
    _clements_body,
    out_type=jax.ShapeDtypeStruct((SC_ROWS, DIM), jnp.float32),
    mesh=plsc.VectorSubcoreMesh(core_axis_name="c", subcore_axis_name="s",
                                num_cores=2, num_subcores=16),
    compiler_params=pltpu.CompilerParams(use_tc_tiling_on_sc=False),
    scratch_types=[
        pltpu.VMEM((ROWS_PER_W, DIM), jnp.float32),
        pltpu.VMEM((ROWS_PER_W, PAD), jnp.float32),
        pltpu.VMEM((ROWS_PER_W, HALF), jnp.float32),
        pltpu.VMEM((NPAIRS, HALF), jnp.float32),
        pltpu.VMEM((NPAIRS, HALF), jnp.float32),
        pltpu.VMEM((NPAIRS, HALF), jnp.float32),
        pltpu.VMEM((NPAIRS, HALF), jnp.float32),
    ],
)


def _mm_body(x_ref, w_ref, o_ref):
    o_ref[...] = lax.dot_general(
        x_ref[...], w_ref[...], (((1,), (0,)), ((), ())),
        preferred_element_type=jnp.float32,
        precision=lax.Precision.HIGHEST)


_apply_tc = pl.pallas_call(
    _mm_body,
    grid=(BATCH // MM_TILE,),
    in_specs=[
        pl.BlockSpec((MM_TILE, DIM), lambda i: (i, 0)),
        pl.BlockSpec((DIM, DIM), lambda i: (0, 0)),
    ],
    out_specs=pl.BlockSpec((MM_TILE, DIM), lambda i: (i, 0)),
    out_shape=jax.ShapeDtypeStruct((BATCH, DIM), jnp.float32),
)

_EYE = np.eye(DIM, dtype=np.float32)


def kernel(x, thetas_even, thetas_odd):
    to_p = jnp.pad(thetas_odd, ((0, 0), (0, 1)))
    ce, se, co, so = _trig(thetas_even, to_p)
    w = _clements_sc(jnp.asarray(_EYE), ce, se, co, so)
    return _apply_tc(x, w)


# speedup vs baseline: 6.2641x; 1.0197x over previous
import numpy as np

import jax
import jax.numpy as jnp
from jax import lax
from jax.experimental import pallas as pl
from jax.experimental.pallas import tpu as pltpu
from jax.experimental.pallas import tpu_sc as plsc

DIM = 256
HALF = DIM // 2
PAD = 144
BATCH = 4096
NPAIRS = 64
NW = 32
L = 16
NG = HALF // L
UNROLL = 2

SC_ROWS = DIM
ROWS_PER_W = SC_ROWS // NW
MM_TILE = 512


def _trig_body(te_ref, top_ref, ce_ref, se_ref, co_ref, so_ref):
    ce_ref[...] = jnp.cos(2.0 * te_ref[...])
    se_ref[...] = jnp.sin(2.0 * te_ref[...])
    co_ref[...] = jnp.cos(2.0 * top_ref[...])
    so_ref[...] = jnp.sin(2.0 * top_ref[...])


_trig = pl.pallas_call(
    _trig_body,
    out_shape=[jax.ShapeDtypeStruct((NPAIRS, HALF), jnp.float32)] * 4,
)


def _take(v, idx):
    return lax.gather(
        v, idx[:, None],
        lax.GatherDimensionNumbers(offset_dims=(), collapsed_slice_dims=(0,),
                                   start_index_map=(0,)),
        slice_sizes=(1,),
        mode=lax.GatherScatterMode.PROMISE_IN_BOUNDS)


def _clements_body(x_hbm, ce_hbm, se_hbm, co_hbm, so_hbm, out_hbm,
                   X, A, B, CE, SE, CO, SO):
    wid = lax.axis_index("s") * 2 + lax.axis_index("c")
    base = wid * ROWS_PER_W
    pltpu.sync_copy(x_hbm.at[pl.ds(base, ROWS_PER_W)], X)
    pltpu.sync_copy(ce_hbm, CE)
    pltpu.sync_copy(se_hbm, SE)
    pltpu.sync_copy(co_hbm, CO)
    pltpu.sync_copy(so_hbm, SO)

    iota = lax.iota(jnp.int32, L)
    idx_a = (2 * iota) & 15
    idx_b = (2 * iota + 1) & 15
    lo_half = iota < 8
    zeros = jnp.zeros((L,), jnp.float32)

    @plsc.parallel_loop(0, ROWS_PER_W, unroll=UNROLL)
    def _deinterleave(r):
        for j in range(NG):
            v0 = X[r, pl.ds(32 * j, L)]
            v1 = X[r, pl.ds(32 * j + 16, L)]
            A[r, pl.ds(L * j, L)] = jnp.where(
                lo_half, _take(v0, idx_a), _take(v1, idx_a))
            B[r, pl.ds(L * j, L)] = jnp.where(
                lo_half, _take(v0, idx_b), _take(v1, idx_b))
        A[r, pl.ds(HALF, L)] = zeros

    def layer_pair(l, carry):
        ces = [CE[l, pl.ds(L * g, L)] for g in range(NG)]
        ses = [SE[l, pl.ds(L * g, L)] for g in range(NG)]

        @plsc.parallel_loop(0, ROWS_PER_W, unroll=UNROLL)
        def _even(r):
            for g in range(NG):
                sl = pl.ds(L * g, L)
                a = A[r, sl]
                b = B[r, sl]
                A[r, sl] = a * ces[g] + b * ses[g]
                B[r, sl] = a * ses[g] - b * ces[g]

        cos_ = [CO[l, pl.ds(L * g, L)] for g in range(NG)]
        sos = [SO[l, pl.ds(L * g, L)] for g in range(NG)]

        @plsc.parallel_loop(0, ROWS_PER_W, unroll=UNROLL)
        def _odd(r):
            for g in range(NG):
                sl = pl.ds(L * g, L)
                sl1 = pl.ds(L * g + 1, L)
                b = B[r, sl]
                a1 = A[r, sl1]
                B[r, sl] = b * cos_[g] + a1 * sos[g]
                A[r, sl1] = b * sos[g] - a1 * cos_[g]

        return carry

    lax.fori_loop(0, NPAIRS, layer_pair, 0)

    idx_h = iota >> 1
    even_lane = (iota & 1) == 0

    @plsc.parallel_loop(0, ROWS_PER_W, unroll=UNROLL)
    def _interleave(r):
        for j in range(NG):
            a = A[r, pl.ds(L * j, L)]
            b = B[r, pl.ds(L * j, L)]
            X[r, pl.ds(32 * j, L)] = jnp.where(
                even_lane, _take(a, idx_h), _take(b, idx_h))
            X[r, pl.ds(32 * j + 16, L)] = jnp.where(
                even_lane, _take(a, 8 + idx_h), _take(b, 8 + idx_h))

    pltpu.sync_copy(X, out_hbm.at[pl.ds(base, ROWS_PER_W)])


_clements_sc = pl.kernel(
    _clements_body,
    out_type=jax.ShapeDtypeStruct((SC_ROWS, DIM), jnp.float32),
    mesh=plsc.VectorSubcoreMesh(core_axis_name="c", subcore_axis_name="s",
                                num_cores=2, num_subcores=16),
    compiler_params=pltpu.CompilerParams(use_tc_tiling_on_sc=False),
    scratch_types=[
        pltpu.VMEM((ROWS_PER_W, DIM), jnp.float32),
        pltpu.VMEM((ROWS_PER_W, PAD), jnp.float32),
        pltpu.VMEM((ROWS_PER_W, HALF), jnp.float32),
        pltpu.VMEM((NPAIRS, HALF), jnp.float32),
        pltpu.VMEM((NPAIRS, HALF), jnp.float32),
        pltpu.VMEM((NPAIRS, HALF), jnp.float32),
        pltpu.VMEM((NPAIRS, HALF), jnp.float32),
    ],
)


def _mm_body(x_ref, w_ref, o_ref):
    o_ref[...] = lax.dot_general(
        x_ref[...], w_ref[...], (((1,), (0,)), ((), ())),
        preferred_element_type=jnp.float32,
        precision=lax.Precision.HIGHEST)


_apply_tc = pl.pallas_call(
    _mm_body,
    grid=(BATCH // MM_TILE,),
    in_specs=[
        pl.BlockSpec((MM_TILE, DIM), lambda i: (i, 0)),
        pl.BlockSpec((DIM, DIM), lambda i: (0, 0)),
    ],
    out_specs=pl.BlockSpec((MM_TILE, DIM), lambda i: (i, 0)),
    out_shape=jax.ShapeDtypeStruct((BATCH, DIM), jnp.float32),
)

_EYE = np.eye(DIM, dtype=np.float32)


def kernel(x, thetas_even, thetas_odd):
    to_p = jnp.pad(thetas_odd, ((0, 0), (0, 1)))
    ce = jnp.cos(2.0 * thetas_even)
    se = jnp.sin(2.0 * thetas_even)
    co = jnp.cos(2.0 * to_p)
    so = jnp.sin(2.0 * to_p)
    w = _clements_sc(jnp.asarray(_EYE), ce, se, co, so)
    return _apply_tc(x, w)


# speedup vs baseline: 6.5440x; 1.0447x over previous
import numpy as np

import jax
import jax.numpy as jnp
from jax import lax
from jax.experimental import pallas as pl
from jax.experimental.pallas import tpu as pltpu
from jax.experimental.pallas import tpu_sc as plsc

DIM = 256
HALF = DIM // 2
PAD = 144
BATCH = 4096
NPAIRS = 64
NW = 32
L = 16
NG = HALF // L
UNROLL = 8

SC_ROWS = DIM
ROWS_PER_W = SC_ROWS // NW
MM_TILE = 512


def _trig_body(te_ref, top_ref, ce_ref, se_ref, co_ref, so_ref):
    ce_ref[...] = jnp.cos(2.0 * te_ref[...])
    se_ref[...] = jnp.sin(2.0 * te_ref[...])
    co_ref[...] = jnp.cos(2.0 * top_ref[...])
    so_ref[...] = jnp.sin(2.0 * top_ref[...])


_trig = pl.pallas_call(
    _trig_body,
    out_shape=[jax.ShapeDtypeStruct((NPAIRS, HALF), jnp.float32)] * 4,
)


def _take(v, idx):
    return lax.gather(
        v, idx[:, None],
        lax.GatherDimensionNumbers(offset_dims=(), collapsed_slice_dims=(0,),
                                   start_index_map=(0,)),
        slice_sizes=(1,),
        mode=lax.GatherScatterMode.PROMISE_IN_BOUNDS)


def _clements_body(x_hbm, ce_hbm, se_hbm, co_hbm, so_hbm, out_hbm,
                   X, A, B, CE, SE, CO, SO):
    wid = lax.axis_index("s") * 2 + lax.axis_index("c")
    base = wid * ROWS_PER_W
    pltpu.sync_copy(x_hbm.at[pl.ds(base, ROWS_PER_W)], X)
    pltpu.sync_copy(ce_hbm, CE)
    pltpu.sync_copy(se_hbm, SE)
    pltpu.sync_copy(co_hbm, CO)
    pltpu.sync_copy(so_hbm, SO)

    iota = lax.iota(jnp.int32, L)
    idx_a = (2 * iota) & 15
    idx_b = (2 * iota + 1) & 15
    lo_half = iota < 8
    zeros = jnp.zeros((L,), jnp.float32)

    @plsc.parallel_loop(0, ROWS_PER_W, unroll=UNROLL)
    def _deinterleave(r):
        for j in range(NG):
            v0 = X[r, pl.ds(32 * j, L)]
            v1 = X[r, pl.ds(32 * j + 16, L)]
            A[r, pl.ds(L * j, L)] = jnp.where(
                lo_half, _take(v0, idx_a), _take(v1, idx_a))
            B[r, pl.ds(L * j, L)] = jnp.where(
                lo_half, _take(v0, idx_b), _take(v1, idx_b))
        A[r, pl.ds(HALF, L)] = zeros

    def layer_pair(l, carry):
        ces = [CE[l, pl.ds(L * g, L)] for g in range(NG)]
        ses = [SE[l, pl.ds(L * g, L)] for g in range(NG)]

        @plsc.parallel_loop(0, ROWS_PER_W, unroll=UNROLL)
        def _even(r):
            for g in range(NG):
                sl = pl.ds(L * g, L)
                a = A[r, sl]
                b = B[r, sl]
                A[r, sl] = a * ces[g] + b * ses[g]
                B[r, sl] = a * ses[g] - b * ces[g]

        cos_ = [CO[l, pl.ds(L * g, L)] for g in range(NG)]
        sos = [SO[l, pl.ds(L * g, L)] for g in range(NG)]

        @plsc.parallel_loop(0, ROWS_PER_W, unroll=UNROLL)
        def _odd(r):
            for g in range(NG):
                sl = pl.ds(L * g, L)
                sl1 = pl.ds(L * g + 1, L)
                b = B[r, sl]
                a1 = A[r, sl1]
                B[r, sl] = b * cos_[g] + a1 * sos[g]
                A[r, sl1] = b * sos[g] - a1 * cos_[g]

        return carry

    lax.fori_loop(0, NPAIRS, layer_pair, 0)

    idx_h = iota >> 1
    even_lane = (iota & 1) == 0

    @plsc.parallel_loop(0, ROWS_PER_W, unroll=UNROLL)
    def _interleave(r):
        for j in range(NG):
            a = A[r, pl.ds(L * j, L)]
            b = B[r, pl.ds(L * j, L)]
            X[r, pl.ds(32 * j, L)] = jnp.where(
                even_lane, _take(a, idx_h), _take(b, idx_h))
            X[r, pl.ds(32 * j + 16, L)] = jnp.where(
                even_lane, _take(a, 8 + idx_h), _take(b, 8 + idx_h))

    pltpu.sync_copy(X, out_hbm.at[pl.ds(base, ROWS_PER_W)])


_clements_sc = pl.kernel(
    _clements_body,
    out_type=jax.ShapeDtypeStruct((SC_ROWS, DIM), jnp.float32),
    mesh=plsc.VectorSubcoreMesh(core_axis_name="c", subcore_axis_name="s",
                                num_cores=2, num_subcores=16),
    compiler_params=pltpu.CompilerParams(use_tc_tiling_on_sc=False),
    scratch_types=[
        pltpu.VMEM((ROWS_PER_W, DIM), jnp.float32),
        pltpu.VMEM((ROWS_PER_W, PAD), jnp.float32),
        pltpu.VMEM((ROWS_PER_W, HALF), jnp.float32),
        pltpu.VMEM((NPAIRS, HALF), jnp.float32),
        pltpu.VMEM((NPAIRS, HALF), jnp.float32),
        pltpu.VMEM((NPAIRS, HALF), jnp.float32),
        pltpu.VMEM((NPAIRS, HALF), jnp.float32),
    ],
)


def _mm_body(x_ref, w_ref, o_ref):
    o_ref[...] = lax.dot_general(
        x_ref[...], w_ref[...], (((1,), (0,)), ((), ())),
        preferred_element_type=jnp.float32,
        precision=lax.Precision.HIGHEST)


_apply_tc = pl.pallas_call(
    _mm_body,
    grid=(BATCH // MM_TILE,),
    in_specs=[
        pl.BlockSpec((MM_TILE, DIM), lambda i: (i, 0)),
        pl.BlockSpec((DIM, DIM), lambda i: (0, 0)),
    ],
    out_specs=pl.BlockSpec((MM_TILE, DIM), lambda i: (i, 0)),
    out_shape=jax.ShapeDtypeStruct((BATCH, DIM), jnp.float32),
)

_EYE = np.eye(DIM, dtype=np.float32)


def kernel(x, thetas_even, thetas_odd):
    to_p = jnp.pad(thetas_odd, ((0, 0), (0, 1)))
    ce, se, co, so = _trig(thetas_even, to_p)
    w = _clements_sc(jnp.asarray(_EYE), ce, se, co, so)
    return _apply_tc(x, w)


# speedup vs baseline: 6.7705x; 1.0346x over previous
import numpy as np

import jax
import jax.numpy as jnp
from jax import lax
from jax.experimental import pallas as pl
from jax.experimental.pallas import tpu as pltpu
from jax.experimental.pallas import tpu_sc as plsc

DIM = 256
HALF = DIM // 2
PAD = 144
BATCH = 4096
NPAIRS = 64
NW = 32
L = 16
NG = HALF // L
UNROLL = 8

SC_ROWS = DIM
ROWS_PER_W = SC_ROWS // NW
MM_TILE = 512


def _trig_body(te_ref, top_ref, ce_ref, se_ref, co_ref, so_ref):
    ce_ref[...] = jnp.cos(2.0 * te_ref[...])
    se_ref[...] = jnp.sin(2.0 * te_ref[...])
    co_ref[...] = jnp.cos(2.0 * top_ref[...])
    so_ref[...] = jnp.sin(2.0 * top_ref[...])


_trig = pl.pallas_call(
    _trig_body,
    out_shape=[jax.ShapeDtypeStruct((NPAIRS, HALF), jnp.float32)] * 4,
)


def _take(v, idx):
    return lax.gather(
        v, idx[:, None],
        lax.GatherDimensionNumbers(offset_dims=(), collapsed_slice_dims=(0,),
                                   start_index_map=(0,)),
        slice_sizes=(1,),
        mode=lax.GatherScatterMode.PROMISE_IN_BOUNDS)


def _clements_body(x_hbm, ce_hbm, se_hbm, co_hbm, so_hbm, out_hbm,
                   X, A, B, CE, SE, CO, SO):
    wid = lax.axis_index("s") * 2 + lax.axis_index("c")
    base = wid * ROWS_PER_W
    pltpu.sync_copy(x_hbm.at[pl.ds(base, ROWS_PER_W)], X)
    pltpu.sync_copy(ce_hbm, CE)
    pltpu.sync_copy(se_hbm, SE)
    pltpu.sync_copy(co_hbm, CO)
    pltpu.sync_copy(so_hbm, SO)

    iota = lax.iota(jnp.int32, L)
    idx_a = (2 * iota) & 15
    idx_b = (2 * iota + 1) & 15
    lo_half = iota < 8
    zeros = jnp.zeros((L,), jnp.float32)

    @plsc.parallel_loop(0, ROWS_PER_W, unroll=UNROLL)
    def _deinterleave(r):
        for j in range(NG):
            v0 = X[r, pl.ds(32 * j, L)]
            v1 = X[r, pl.ds(32 * j + 16, L)]
            A[r, pl.ds(L * j, L)] = jnp.where(
                lo_half, _take(v0, idx_a), _take(v1, idx_a))
            B[r, pl.ds(L * j, L)] = jnp.where(
                lo_half, _take(v0, idx_b), _take(v1, idx_b))
        A[r, pl.ds(HALF, L)] = zeros

    def layer_pair(l, carry):
        ces = [CE[l, pl.ds(L * g, L)] for g in range(NG)]
        ses = [SE[l, pl.ds(L * g, L)] for g in range(NG)]

        @plsc.parallel_loop(0, ROWS_PER_W, unroll=UNROLL)
        def _even(r):
            for g in range(NG):
                sl = pl.ds(L * g, L)
                a = A[r, sl]
                b = B[r, sl]
                A[r, sl] = a * ces[g] + b * ses[g]
                B[r, sl] = a * ses[g] - b * ces[g]

        cos_ = [CO[l, pl.ds(L * g, L)] for g in range(NG)]
        sos = [SO[l, pl.ds(L * g, L)] for g in range(NG)]

        @plsc.parallel_loop(0, ROWS_PER_W, unroll=UNROLL)
        def _odd(r):
            for g in range(NG):
                sl = pl.ds(L * g, L)
                sl1 = pl.ds(L * g + 1, L)
                b = B[r, sl]
                a1 = A[r, sl1]
                B[r, sl] = b * cos_[g] + a1 * sos[g]
                A[r, sl1] = b * sos[g] - a1 * cos_[g]

        return carry

    lax.fori_loop(0, NPAIRS, layer_pair, 0)

    idx_h = iota >> 1
    even_lane = (iota & 1) == 0

    @plsc.parallel_loop(0, ROWS_PER_W, unroll=UNROLL)
    def _interleave(r):
        for j in range(NG):
            a = A[r, pl.ds(L * j, L)]
            b = B[r, pl.ds(L * j, L)]
            X[r, pl.ds(32 * j, L)] = jnp.where(
                even_lane, _take(a, idx_h), _take(b, idx_h))
            X[r, pl.ds(32 * j + 16, L)] = jnp.where(
                even_lane, _take(a, 8 + idx_h), _take(b, 8 + idx_h))

    pltpu.sync_copy(X, out_hbm.at[pl.ds(base, ROWS_PER_W)])


_clements_sc = pl.kernel(
    _clements_body,
    out_type=jax.ShapeDtypeStruct((SC_ROWS, DIM), jnp.float32),
    mesh=plsc.VectorSubcoreMesh(core_axis_name="c", subcore_axis_name="s",
                                num_cores=2, num_subcores=16),
    compiler_params=pltpu.CompilerParams(use_tc_tiling_on_sc=False),
    scratch_types=[
        pltpu.VMEM((ROWS_PER_W, DIM), jnp.float32),
        pltpu.VMEM((ROWS_PER_W, PAD), jnp.float32),
        pltpu.VMEM((ROWS_PER_W, HALF), jnp.float32),
        pltpu.VMEM((NPAIRS, HALF), jnp.float32),
        pltpu.VMEM((NPAIRS, HALF), jnp.float32),
        pltpu.VMEM((NPAIRS, HALF), jnp.float32),
        pltpu.VMEM((NPAIRS, HALF), jnp.float32),
    ],
)


def _mm_body(x_ref, w_ref, o_ref):
    o_ref[...] = lax.dot_general(
        x_ref[...], w_ref[...], (((1,), (0,)), ((), ())),
        preferred_element_type=jnp.float32,
        precision=lax.Precision.DEFAULT)


_apply_tc = pl.pallas_call(
    _mm_body,
    grid=(BATCH // MM_TILE,),
    in_specs=[
        pl.BlockSpec((MM_TILE, DIM), lambda i: (i, 0)),
        pl.BlockSpec((DIM, DIM), lambda i: (0, 0)),
    ],
    out_specs=pl.BlockSpec((MM_TILE, DIM), lambda i: (i, 0)),
    out_shape=jax.ShapeDtypeStruct((BATCH, DIM), jnp.float32),
)

_EYE = np.eye(DIM, dtype=np.float32)


def kernel(x, thetas_even, thetas_odd):
    to_p = jnp.pad(thetas_odd, ((0, 0), (0, 1)))
    ce, se, co, so = _trig(thetas_even, to_p)
    w = _clements_sc(jnp.asarray(_EYE), ce, se, co, so)
    return _apply_tc(x, w)


# speedup vs baseline: 7.1056x; 1.0495x over previous
import numpy as np

import jax
import jax.numpy as jnp
from jax import lax
from jax.experimental import pallas as pl
from jax.experimental.pallas import tpu as pltpu
from jax.experimental.pallas import tpu_sc as plsc

DIM = 256
HALF = DIM // 2
PAD = 144
BATCH = 4096
NPAIRS = 64
NW = 32
L = 16
NG = HALF // L
UNROLL = 8

SC_ROWS = DIM
ROWS_PER_W = SC_ROWS // NW
MM_TILE = 1024


def _trig_body(te_ref, top_ref, ce_ref, se_ref, co_ref, so_ref):
    ce_ref[...] = jnp.cos(2.0 * te_ref[...])
    se_ref[...] = jnp.sin(2.0 * te_ref[...])
    co_ref[...] = jnp.cos(2.0 * top_ref[...])
    so_ref[...] = jnp.sin(2.0 * top_ref[...])


_trig = pl.pallas_call(
    _trig_body,
    out_shape=[jax.ShapeDtypeStruct((NPAIRS, HALF), jnp.float32)] * 4,
)


def _take(v, idx):
    return lax.gather(
        v, idx[:, None],
        lax.GatherDimensionNumbers(offset_dims=(), collapsed_slice_dims=(0,),
                                   start_index_map=(0,)),
        slice_sizes=(1,),
        mode=lax.GatherScatterMode.PROMISE_IN_BOUNDS)


def _clements_body(x_hbm, ce_hbm, se_hbm, co_hbm, so_hbm, out_hbm,
                   X, A, B, CE, SE, CO, SO):
    wid = lax.axis_index("s") * 2 + lax.axis_index("c")
    base = wid * ROWS_PER_W
    pltpu.sync_copy(x_hbm.at[pl.ds(base, ROWS_PER_W)], X)
    pltpu.sync_copy(ce_hbm, CE)
    pltpu.sync_copy(se_hbm, SE)
    pltpu.sync_copy(co_hbm, CO)
    pltpu.sync_copy(so_hbm, SO)

    iota = lax.iota(jnp.int32, L)
    idx_a = (2 * iota) & 15
    idx_b = (2 * iota + 1) & 15
    lo_half = iota < 8
    zeros = jnp.zeros((L,), jnp.float32)

    @plsc.parallel_loop(0, ROWS_PER_W, unroll=UNROLL)
    def _deinterleave(r):
        for j in range(NG):
            v0 = X[r, pl.ds(32 * j, L)]
            v1 = X[r, pl.ds(32 * j + 16, L)]
            A[r, pl.ds(L * j, L)] = jnp.where(
                lo_half, _take(v0, idx_a), _take(v1, idx_a))
            B[r, pl.ds(L * j, L)] = jnp.where(
                lo_half, _take(v0, idx_b), _take(v1, idx_b))
        A[r, pl.ds(HALF, L)] = zeros

    def layer_pair(l, carry):
        ces = [CE[l, pl.ds(L * g, L)] for g in range(NG)]
        ses = [SE[l, pl.ds(L * g, L)] for g in range(NG)]

        @plsc.parallel_loop(0, ROWS_PER_W, unroll=UNROLL)
        def _even(r):
            for g in range(NG):
                sl = pl.ds(L * g, L)
                a = A[r, sl]
                b = B[r, sl]
                A[r, sl] = a * ces[g] + b * ses[g]
                B[r, sl] = a * ses[g] - b * ces[g]

        cos_ = [CO[l, pl.ds(L * g, L)] for g in range(NG)]
        sos = [SO[l, pl.ds(L * g, L)] for g in range(NG)]

        @plsc.parallel_loop(0, ROWS_PER_W, unroll=UNROLL)
        def _odd(r):
            for g in range(NG):
                sl = pl.ds(L * g, L)
                sl1 = pl.ds(L * g + 1, L)
                b = B[r, sl]
                a1 = A[r, sl1]
                B[r, sl] = b * cos_[g] + a1 * sos[g]
                A[r, sl1] = b * sos[g] - a1 * cos_[g]

        return carry

    lax.fori_loop(0, NPAIRS, layer_pair, 0)

    idx_h = iota >> 1
    even_lane = (iota & 1) == 0

    @plsc.parallel_loop(0, ROWS_PER_W, unroll=UNROLL)
    def _interleave(r):
        for j in range(NG):
            a = A[r, pl.ds(L * j, L)]
            b = B[r, pl.ds(L * j, L)]
            X[r, pl.ds(32 * j, L)] = jnp.where(
                even_lane, _take(a, idx_h), _take(b, idx_h))
            X[r, pl.ds(32 * j + 16, L)] = jnp.where(
                even_lane, _take(a, 8 + idx_h), _take(b, 8 + idx_h))

    pltpu.sync_copy(X, out_hbm.at[pl.ds(base, ROWS_PER_W)])


_clements_sc = pl.kernel(
    _clements_body,
    out_type=jax.ShapeDtypeStruct((SC_ROWS, DIM), jnp.float32),
    mesh=plsc.VectorSubcoreMesh(core_axis_name="c", subcore_axis_name="s",
                                num_cores=2, num_subcores=16),
    compiler_params=pltpu.CompilerParams(use_tc_tiling_on_sc=False),
    scratch_types=[
        pltpu.VMEM((ROWS_PER_W, DIM), jnp.float32),
        pltpu.VMEM((ROWS_PER_W, PAD), jnp.float32),
        pltpu.VMEM((ROWS_PER_W, HALF), jnp.float32),
        pltpu.VMEM((NPAIRS, HALF), jnp.float32),
        pltpu.VMEM((NPAIRS, HALF), jnp.float32),
        pltpu.VMEM((NPAIRS, HALF), jnp.float32),
        pltpu.VMEM((NPAIRS, HALF), jnp.float32),
    ],
)


def _mm_body(x_ref, w_ref, o_ref):
    o_ref[...] = lax.dot_general(
        x_ref[...], w_ref[...], (((1,), (0,)), ((), ())),
        preferred_element_type=jnp.float32,
        precision=lax.Precision.DEFAULT)


_apply_tc = pl.pallas_call(
    _mm_body,
    grid=(BATCH // MM_TILE,),
    in_specs=[
        pl.BlockSpec((MM_TILE, DIM), lambda i: (i, 0)),
        pl.BlockSpec((DIM, DIM), lambda i: (0, 0)),
    ],
    out_specs=pl.BlockSpec((MM_TILE, DIM), lambda i: (i, 0)),
    out_shape=jax.ShapeDtypeStruct((BATCH, DIM), jnp.float32),
)

_EYE = np.eye(DIM, dtype=np.float32)


def kernel(x, thetas_even, thetas_odd):
    to_p = jnp.pad(thetas_odd, ((0, 0), (0, 1)))
    ce, se, co, so = _trig(thetas_even, to_p)
    w = _clements_sc(jnp.asarray(_EYE), ce, se, co, so)
    return _apply_tc(x, w)
